# baseline (speedup 1.0000x reference)
import functools

import jax
import jax.numpy as jnp
from jax import lax
from jax.experimental import pallas as pl
from jax.experimental.pallas import tpu as pltpu
from jax.experimental.pallas import tpu_sc as plsc

_NC, _NS, _L = 2, 16, 16
_NW = _NC * _NS
_MAX_SLIP = 0.4
_MAX_GUESS = 0.4
_T_INV = 1.0 / 50.0
_SW = 8


def _sigmoid(x):
    return 1.0 / (1.0 + jnp.exp(-x))


def _make_sc_kernel(B, H, S, C):
    RPW = B // _NW
    NCHUNK = RPW // C
    GRP = C // _L
    HB = H // _L
    OFF_W1 = 3 * H
    OFF_B1 = OFF_W1 + 48
    OFF_W2 = OFF_B1 + 16
    OFF_B2 = OFF_W2 + 16
    WPACK = OFF_B2 + 16

    mesh = plsc.VectorSubcoreMesh(core_axis_name="c", subcore_axis_name="s",
                                  num_cores=_NC, num_subcores=_NS)

    @functools.partial(
        pl.kernel,
        out_type=jax.ShapeDtypeStruct((B,), jnp.float32),
        mesh=mesh,
        scratch_types=[
            pltpu.VMEM((C,), jnp.int32),
            pltpu.VMEM((C,), jnp.int32),
            pltpu.VMEM((C, H), jnp.float32),
            pltpu.VMEM((C, S, H), jnp.float32),
            pltpu.VMEM((C, H), jnp.float32),
            pltpu.VMEM((C, _SW), jnp.float32),
            pltpu.VMEM((C, _SW), jnp.float32),
            pltpu.VMEM((WPACK,), jnp.float32),
            pltpu.VMEM((_L, _L + 1), jnp.float32),
            pltpu.VMEM((_L, _L + 1), jnp.float32),
            pltpu.VMEM((C,), jnp.float32),
            pltpu.SemaphoreType.DMA,
        ],
        compiler_params=pltpu.CompilerParams(
            needs_layout_passes=False, use_tc_tiling_on_sc=False),
    )
    def k(user, item, knowledge, theta_t, us_t, it_t, q_t,
          wpack, out, idx_u, idx_i, th_v, q_v, kn_v, us_v, it_v,
          wp_v, st0_v, st1_v, out_v, sem):
        wid = lax.axis_index("s") * _NC + lax.axis_index("c")
        pltpu.sync_copy(wpack, wp_v)
        lanes = lax.iota(jnp.int32, _L)
        z16 = jnp.zeros((_L,), jnp.int32)
        o16 = jnp.ones((_L,), jnp.int32)
        aw = [[wp_v[pl.ds(j * H + kk * _L, _L)] for kk in range(HB)]
              for j in range(3)]
        w1c = [wp_v[pl.ds(OFF_W1 + 16 * kk, 16)] for kk in range(3)]
        b1v = wp_v[pl.ds(OFF_B1, 16)]
        w2v = wp_v[pl.ds(OFF_W2, 16)]
        b2s = wp_v[pl.ds(OFF_B2, 16)][0]
        fzero = jnp.zeros((_L,), jnp.float32)

        def chunk_body(chunk, _):
            base = wid * RPW + chunk * C
            pltpu.sync_copy(user.at[pl.ds(base, C)], idx_u)
            pltpu.sync_copy(item.at[pl.ds(base, C)], idx_i)
            cps = (
                pltpu.async_copy(theta_t.at[idx_u], th_v, sem),
                pltpu.async_copy(q_t.at[idx_i], q_v, sem),
                pltpu.async_copy(knowledge.at[pl.ds(base, C)], kn_v, sem),
                pltpu.async_copy(us_t.at[idx_u], us_v, sem),
                pltpu.async_copy(it_t.at[idx_i], it_v, sem),
            )
            for cp in cps:
                cp.wait()

            def group(g, _):
                r0 = g * _L
                ridx = r0 + lanes
                a0 = plsc.load_gather(us_v, [ridx, z16])
                a1 = plsc.load_gather(us_v, [ridx, o16])
                a2 = plsc.load_gather(us_v, [ridx, z16 + 2])
                acc = fzero
                for j in range(16):
                    hj = w1c[0][j] * a0 + w1c[1][j] * a1 + w1c[2][j] * a2 + b1v[j]
                    acc = acc + w2v[j] * jnp.maximum(hj, 0.0)
                af = _sigmoid(acc + b2s)
                f0 = af * a0
                f1 = af * a1
                f2 = af * a2

                for r in range(_L):
                    row = r0 + r
                    s0, s1, s2 = f0[r], f1[r], f2[r]
                    m0p = fzero
                    m1p = fzero
                    for kk in range(HB):
                        c0 = kk * _L
                        th = (th_v[row, pl.ds(c0, _L)]
                              + s0 * aw[0][kk] + s1 * aw[1][kk] + s2 * aw[2][kk])
                        sg = _sigmoid(th) - 0.5
                        kq = kn_v[row, pl.ds(c0, _L)] * sg
                        m0p = m0p + kq * _sigmoid(q_v[row, 0, pl.ds(c0, _L)])
                        m1p = m1p + kq * _sigmoid(q_v[row, 1, pl.ds(c0, _L)])
                    st0_v[r, pl.ds(0, _L)] = m0p
                    st1_v[r, pl.ds(0, _L)] = m1p

                m0 = fzero
                m1 = fzero
                for j in range(_L):
                    jj = jnp.full((_L,), j, jnp.int32)
                    m0 = m0 + plsc.load_gather(st0_v, [lanes, jj])
                    m1 = m1 + plsc.load_gather(st1_v, [lanes, jj])
                p0 = _sigmoid(m0 * _T_INV)
                p1 = _sigmoid(m1 * _T_INV)
                slip = _sigmoid(plsc.load_gather(it_v, [ridx, z16])) * _MAX_SLIP
                guess = _sigmoid(plsc.load_gather(it_v, [ridx, o16])) * _MAX_GUESS
                w0 = plsc.load_gather(it_v, [ridx, z16 + 2])
                w1 = plsc.load_gather(it_v, [ridx, z16 + 3])
                sp0 = _sigmoid(w0 - w1)
                span = 1.0 - slip - guess
                cc0 = guess + span * p0
                cc1 = guess + span * p1
                out_v[pl.ds(r0, _L)] = cc1 + sp0 * (cc0 - cc1)
                return 0

            lax.fori_loop(0, GRP, group, 0)
            pltpu.sync_copy(out_v, out.at[pl.ds(base, C)])
            return 0

        lax.fori_loop(0, NCHUNK, chunk_body, 0)

    return k


def kernel(user, item, knowledge, theta_table, affect_table, slip_table,
           guess_table, strategy_weights, strategy_q, affect_weight,
           W1, b1, W2, b2):
    B = user.shape[0]
    H = theta_table.shape[1]
    S = strategy_weights.shape[1]
    U = affect_table.shape[0]
    I = slip_table.shape[0]
    us_t = jnp.concatenate(
        [affect_table, jnp.zeros((U, _SW - 3), jnp.float32)], axis=1)
    it_t = jnp.concatenate(
        [slip_table, guess_table, strategy_weights,
         jnp.zeros((I, _SW - 2 - S), jnp.float32)], axis=1)
    wpack = jnp.concatenate([
        affect_weight.reshape(-1),
        W1.T.reshape(-1),
        b1.reshape(-1),
        W2.reshape(-1),
        b2.reshape(-1),
        jnp.zeros((15,), jnp.float32),
    ])
    k = _make_sc_kernel(B, H, S, C=128)
    return k(user.astype(jnp.int32), item.astype(jnp.int32), knowledge,
             theta_table, us_t, it_t, strategy_q, wpack)


# speedup vs baseline: 1.7601x; 1.7601x over previous
import functools

import jax
import jax.numpy as jnp
from jax import lax
from jax.experimental import pallas as pl
from jax.experimental.pallas import tpu as pltpu
from jax.experimental.pallas import tpu_sc as plsc

_NC, _NS, _L = 2, 16, 16
_NW = _NC * _NS
_MAX_SLIP = 0.4
_MAX_GUESS = 0.4
_T_INV = 1.0 / 50.0
_SW = 8


def _sigmoid(x):
    return 1.0 / (1.0 + jnp.exp(-x))


def _make_sc_kernel(B, H, S, C):
    RPW = B // _NW
    NCHUNK = RPW // C
    GRP = C // _L
    HB = H // _L
    OFF_W1 = 3 * H
    OFF_B1 = OFF_W1 + 48
    OFF_W2 = OFF_B1 + 16
    OFF_B2 = OFF_W2 + 16
    WPACK = OFF_B2 + 16

    mesh = plsc.VectorSubcoreMesh(core_axis_name="c", subcore_axis_name="s",
                                  num_cores=_NC, num_subcores=_NS)

    @functools.partial(
        pl.kernel,
        out_type=jax.ShapeDtypeStruct((B,), jnp.float32),
        mesh=mesh,
        scratch_types=[
            pltpu.VMEM((C,), jnp.int32),
            pltpu.VMEM((C,), jnp.int32),
            pltpu.VMEM((C, H), jnp.float32),
            pltpu.VMEM((C, S, H), jnp.float32),
            pltpu.VMEM((C, H), jnp.float32),
            pltpu.VMEM((C, _SW), jnp.float32),
            pltpu.VMEM((C, _SW), jnp.float32),
            pltpu.VMEM((WPACK,), jnp.float32),
            pltpu.VMEM((_L, _L + 1), jnp.float32),
            pltpu.VMEM((_L, _L + 1), jnp.float32),
            pltpu.VMEM((3, _L), jnp.float32),
            pltpu.VMEM((C,), jnp.float32),
            pltpu.SemaphoreType.DMA,
        ],
        compiler_params=pltpu.CompilerParams(
            needs_layout_passes=False, use_tc_tiling_on_sc=False),
    )
    def k(user, item, knowledge, theta_t, us_t, it_t, q_t,
          wpack, out, idx_u, idx_i, th_v, q_v, kn_v, us_v, it_v,
          wp_v, st0_v, st1_v, fb_v, out_v, sem):
        wid = lax.axis_index("s") * _NC + lax.axis_index("c")
        pltpu.sync_copy(wpack, wp_v)
        lanes = lax.iota(jnp.int32, _L)
        z16 = jnp.zeros((_L,), jnp.int32)
        o16 = jnp.ones((_L,), jnp.int32)
        aw = [[wp_v[pl.ds(j * H + kk * _L, _L)] for kk in range(HB)]
              for j in range(3)]
        w1c = [wp_v[pl.ds(OFF_W1 + 16 * kk, 16)] for kk in range(3)]
        b1v = wp_v[pl.ds(OFF_B1, 16)]
        w2v = wp_v[pl.ds(OFF_W2, 16)]
        b2s = wp_v[pl.ds(OFF_B2, 16)][0]
        fzero = jnp.zeros((_L,), jnp.float32)

        def chunk_body(chunk, _):
            base = wid * RPW + chunk * C
            pltpu.sync_copy(user.at[pl.ds(base, C)], idx_u)
            pltpu.sync_copy(item.at[pl.ds(base, C)], idx_i)
            cps = (
                pltpu.async_copy(theta_t.at[idx_u], th_v, sem),
                pltpu.async_copy(q_t.at[idx_i], q_v, sem),
                pltpu.async_copy(knowledge.at[pl.ds(base, C)], kn_v, sem),
                pltpu.async_copy(us_t.at[idx_u], us_v, sem),
                pltpu.async_copy(it_t.at[idx_i], it_v, sem),
            )
            for cp in cps:
                cp.wait()

            def group(g, _):
                r0 = g * _L
                ridx = r0 + lanes
                a0 = plsc.load_gather(us_v, [ridx, z16])
                a1 = plsc.load_gather(us_v, [ridx, o16])
                a2 = plsc.load_gather(us_v, [ridx, z16 + 2])
                acc = fzero
                for j in range(16):
                    hj = w1c[0][j] * a0 + w1c[1][j] * a1 + w1c[2][j] * a2 + b1v[j]
                    acc = acc + w2v[j] * jnp.maximum(hj, 0.0)
                af = _sigmoid(acc + b2s)
                fb_v[0, pl.ds(0, _L)] = af * a0
                fb_v[1, pl.ds(0, _L)] = af * a1
                fb_v[2, pl.ds(0, _L)] = af * a2

                @plsc.parallel_loop(0, _L, 1, unroll=2)
                def _row(r):
                    row = r0 + r
                    rv = jnp.full((_L,), r, jnp.int32)
                    s0 = plsc.load_gather(fb_v, [z16, rv])
                    s1 = plsc.load_gather(fb_v, [o16, rv])
                    s2 = plsc.load_gather(fb_v, [z16 + 2, rv])
                    m0p = fzero
                    m1p = fzero
                    for kk in range(HB):
                        c0 = kk * _L
                        th = (th_v[row, pl.ds(c0, _L)]
                              + s0 * aw[0][kk] + s1 * aw[1][kk] + s2 * aw[2][kk])
                        sg = _sigmoid(th) - 0.5
                        kq = kn_v[row, pl.ds(c0, _L)] * sg
                        m0p = m0p + kq * _sigmoid(q_v[row, 0, pl.ds(c0, _L)])
                        m1p = m1p + kq * _sigmoid(q_v[row, 1, pl.ds(c0, _L)])
                    st0_v[r, pl.ds(0, _L)] = m0p
                    st1_v[r, pl.ds(0, _L)] = m1p

                m0 = fzero
                m1 = fzero
                for j in range(_L):
                    jj = jnp.full((_L,), j, jnp.int32)
                    m0 = m0 + plsc.load_gather(st0_v, [lanes, jj])
                    m1 = m1 + plsc.load_gather(st1_v, [lanes, jj])
                p0 = _sigmoid(m0 * _T_INV)
                p1 = _sigmoid(m1 * _T_INV)
                slip = _sigmoid(plsc.load_gather(it_v, [ridx, z16])) * _MAX_SLIP
                guess = _sigmoid(plsc.load_gather(it_v, [ridx, o16])) * _MAX_GUESS
                w0 = plsc.load_gather(it_v, [ridx, z16 + 2])
                w1 = plsc.load_gather(it_v, [ridx, z16 + 3])
                sp0 = _sigmoid(w0 - w1)
                span = 1.0 - slip - guess
                cc0 = guess + span * p0
                cc1 = guess + span * p1
                out_v[pl.ds(r0, _L)] = cc1 + sp0 * (cc0 - cc1)
                return 0

            lax.fori_loop(0, GRP, group, 0)
            pltpu.sync_copy(out_v, out.at[pl.ds(base, C)])
            return 0

        lax.fori_loop(0, NCHUNK, chunk_body, 0)

    return k


def kernel(user, item, knowledge, theta_table, affect_table, slip_table,
           guess_table, strategy_weights, strategy_q, affect_weight,
           W1, b1, W2, b2):
    B = user.shape[0]
    H = theta_table.shape[1]
    S = strategy_weights.shape[1]
    U = affect_table.shape[0]
    I = slip_table.shape[0]
    us_t = jnp.concatenate(
        [affect_table, jnp.zeros((U, _SW - 3), jnp.float32)], axis=1)
    it_t = jnp.concatenate(
        [slip_table, guess_table, strategy_weights,
         jnp.zeros((I, _SW - 2 - S), jnp.float32)], axis=1)
    wpack = jnp.concatenate([
        affect_weight.reshape(-1),
        W1.T.reshape(-1),
        b1.reshape(-1),
        W2.reshape(-1),
        b2.reshape(-1),
        jnp.zeros((15,), jnp.float32),
    ])
    k = _make_sc_kernel(B, H, S, C=128)
    return k(user.astype(jnp.int32), item.astype(jnp.int32), knowledge,
             theta_table, us_t, it_t, strategy_q, wpack)


# speedup vs baseline: 1.9928x; 1.1322x over previous
import functools

import jax
import jax.numpy as jnp
from jax import lax
from jax.experimental import pallas as pl
from jax.experimental.pallas import tpu as pltpu
from jax.experimental.pallas import tpu_sc as plsc

_NC, _NS, _L = 2, 16, 16
_NW = _NC * _NS
_MAX_SLIP = 0.4
_MAX_GUESS = 0.4
_T_INV = 1.0 / 50.0
_SW = 8


def _sigmoid(x):
    return 1.0 / (1.0 + jnp.exp(-x))


def _make_sc_kernel(B, H, S, C, AF_ROWS):
    RPW = B // _NW
    NCHUNK = RPW // C
    GRP = C // _L
    HB = H // _L
    OFF_W1 = 3 * H
    OFF_B1 = OFF_W1 + 48
    OFF_W2 = OFF_B1 + 16
    OFF_B2 = OFF_W2 + 16
    WPACK = OFF_B2 + 16

    mesh = plsc.VectorSubcoreMesh(core_axis_name="c", subcore_axis_name="s",
                                  num_cores=_NC, num_subcores=_NS)

    @functools.partial(
        pl.kernel,
        out_type=jax.ShapeDtypeStruct((B,), jnp.float32),
        mesh=mesh,
        scratch_types=[
            pltpu.VMEM((C,), jnp.int32),
            pltpu.VMEM((C,), jnp.int32),
            pltpu.VMEM((2 * C,), jnp.int32),
            pltpu.VMEM((C,), jnp.int32),
            pltpu.VMEM((C,), jnp.int32),
            pltpu.VMEM((C, H), jnp.float32),
            pltpu.VMEM((C, S, H), jnp.float32),
            pltpu.VMEM((C, H), jnp.float32),
            pltpu.VMEM((2 * C, _SW), jnp.float32),
            pltpu.VMEM((C, _SW), jnp.float32),
            pltpu.VMEM((C, _SW), jnp.float32),
            pltpu.VMEM((C, _SW), jnp.float32),
            pltpu.VMEM((WPACK,), jnp.float32),
            pltpu.VMEM((_L, _L + 1), jnp.float32),
            pltpu.VMEM((_L, _L + 1), jnp.float32),
            pltpu.VMEM((3, _L), jnp.float32),
            pltpu.VMEM((C,), jnp.float32),
            pltpu.SemaphoreType.DMA,
        ],
        compiler_params=pltpu.CompilerParams(
            needs_layout_passes=False, use_tc_tiling_on_sc=False),
    )
    def k(user, item, knowledge, theta_t, af8_t, sl8_t, gu8_t, sw8_t, q_t,
          wpack, out, idx_u, idx_i, aidx_v, sidx_v, widx_v,
          th_v, q_v, kn_v, af_v, sl_v, gu_v, sw_v,
          wp_v, st0_v, st1_v, fb_v, out_v, sem):
        wid = lax.axis_index("s") * _NC + lax.axis_index("c")
        pltpu.sync_copy(wpack, wp_v)
        lanes = lax.iota(jnp.int32, _L)
        z16 = jnp.zeros((_L,), jnp.int32)
        o16 = jnp.ones((_L,), jnp.int32)
        aw = [[wp_v[pl.ds(j * H + kk * _L, _L)] for kk in range(HB)]
              for j in range(3)]
        w1c = [wp_v[pl.ds(OFF_W1 + 16 * kk, 16)] for kk in range(3)]
        b1v = wp_v[pl.ds(OFF_B1, 16)]
        w2v = wp_v[pl.ds(OFF_W2, 16)]
        b2s = wp_v[pl.ds(OFF_B2, 16)][0]
        fzero = jnp.zeros((_L,), jnp.float32)

        def chunk_body(chunk, _):
            base = wid * RPW + chunk * C
            pltpu.sync_copy(user.at[pl.ds(base, C)], idx_u)
            pltpu.sync_copy(item.at[pl.ds(base, C)], idx_i)
            cps_a = (
                pltpu.async_copy(theta_t.at[idx_u], th_v, sem),
                pltpu.async_copy(q_t.at[idx_i], q_v, sem),
                pltpu.async_copy(knowledge.at[pl.ds(base, C)], kn_v, sem),
            )
            for g in range(GRP):
                sl = pl.ds(g * _L, _L)
                iu3 = idx_u[sl] * 3
                r0v = lax.shift_right_logical(iu3, 1 + 2)
                aidx_v[sl] = r0v
                aidx_v[pl.ds(C + g * _L, _L)] = jnp.minimum(r0v + 1, AF_ROWS - 1)
                ii = idx_i[sl]
                sidx_v[sl] = lax.shift_right_logical(ii, 3)
                widx_v[sl] = lax.shift_right_logical(ii, 2)
            cps_b = (
                pltpu.async_copy(af8_t.at[aidx_v], af_v, sem),
                pltpu.async_copy(sl8_t.at[sidx_v], sl_v, sem),
                pltpu.async_copy(gu8_t.at[sidx_v], gu_v, sem),
                pltpu.async_copy(sw8_t.at[widx_v], sw_v, sem),
            )
            for cp in cps_a + cps_b:
                cp.wait()

            def group(g, _):
                r0 = g * _L
                ridx = r0 + lanes
                iu3 = idx_u[pl.ds(r0, _L)] * 3
                ii = idx_i[pl.ds(r0, _L)]
                oa = jnp.bitwise_and(iu3, 7)
                avals = []
                for j in range(3):
                    t = oa + j
                    rowv = ridx + jnp.where(t >= _SW, C, 0)
                    avals.append(plsc.load_gather(
                        af_v, [rowv, jnp.bitwise_and(t, 7)]))
                a0, a1, a2 = avals
                acc = fzero
                for j in range(16):
                    hj = w1c[0][j] * a0 + w1c[1][j] * a1 + w1c[2][j] * a2 + b1v[j]
                    acc = acc + w2v[j] * jnp.maximum(hj, 0.0)
                af = _sigmoid(acc + b2s)
                fb_v[0, pl.ds(0, _L)] = af * a0
                fb_v[1, pl.ds(0, _L)] = af * a1
                fb_v[2, pl.ds(0, _L)] = af * a2

                @plsc.parallel_loop(0, _L, 1, unroll=2)
                def _row(r):
                    row = r0 + r
                    rv = jnp.full((_L,), r, jnp.int32)
                    s0 = plsc.load_gather(fb_v, [z16, rv])
                    s1 = plsc.load_gather(fb_v, [o16, rv])
                    s2 = plsc.load_gather(fb_v, [z16 + 2, rv])
                    m0p = fzero
                    m1p = fzero
                    for kk in range(HB):
                        c0 = kk * _L
                        th = (th_v[row, pl.ds(c0, _L)]
                              + s0 * aw[0][kk] + s1 * aw[1][kk] + s2 * aw[2][kk])
                        sg = _sigmoid(th) - 0.5
                        kq = kn_v[row, pl.ds(c0, _L)] * sg
                        m0p = m0p + kq * _sigmoid(q_v[row, 0, pl.ds(c0, _L)])
                        m1p = m1p + kq * _sigmoid(q_v[row, 1, pl.ds(c0, _L)])
                    st0_v[r, pl.ds(0, _L)] = m0p
                    st1_v[r, pl.ds(0, _L)] = m1p

                m0 = fzero
                m1 = fzero
                for j in range(_L):
                    jj = jnp.full((_L,), j, jnp.int32)
                    m0 = m0 + plsc.load_gather(st0_v, [lanes, jj])
                    m1 = m1 + plsc.load_gather(st1_v, [lanes, jj])
                p0 = _sigmoid(m0 * _T_INV)
                p1 = _sigmoid(m1 * _T_INV)
                os_ = jnp.bitwise_and(ii, 7)
                ow = jnp.bitwise_and(ii, 3) * 2
                slip = _sigmoid(plsc.load_gather(sl_v, [ridx, os_])) * _MAX_SLIP
                guess = _sigmoid(plsc.load_gather(gu_v, [ridx, os_])) * _MAX_GUESS
                w0 = plsc.load_gather(sw_v, [ridx, ow])
                w1 = plsc.load_gather(sw_v, [ridx, ow + 1])
                sp0 = _sigmoid(w0 - w1)
                span = 1.0 - slip - guess
                cc0 = guess + span * p0
                cc1 = guess + span * p1
                out_v[pl.ds(r0, _L)] = cc1 + sp0 * (cc0 - cc1)
                return 0

            lax.fori_loop(0, GRP, group, 0)
            pltpu.sync_copy(out_v, out.at[pl.ds(base, C)])
            return 0

        lax.fori_loop(0, NCHUNK, chunk_body, 0)

    return k


def kernel(user, item, knowledge, theta_table, affect_table, slip_table,
           guess_table, strategy_weights, strategy_q, affect_weight,
           W1, b1, W2, b2):
    B = user.shape[0]
    H = theta_table.shape[1]
    S = strategy_weights.shape[1]
    U = affect_table.shape[0]
    I = slip_table.shape[0]
    af8 = affect_table.reshape(U * 3 // _SW, _SW)
    sl8 = slip_table.reshape(I // _SW, _SW)
    gu8 = guess_table.reshape(I // _SW, _SW)
    sw8 = strategy_weights.reshape(I * S // _SW, _SW)
    wpack = jnp.concatenate([
        affect_weight.reshape(-1),
        W1.T.reshape(-1),
        b1.reshape(-1),
        W2.reshape(-1),
        b2.reshape(-1),
        jnp.zeros((15,), jnp.float32),
    ])
    k = _make_sc_kernel(B, H, S, 128, af8.shape[0])
    return k(user.astype(jnp.int32), item.astype(jnp.int32), knowledge,
             theta_table, af8, sl8, gu8, sw8, strategy_q, wpack)


# speedup vs baseline: 4.2160x; 2.1156x over previous
import functools

import jax
import jax.numpy as jnp
from jax import lax
from jax.experimental import pallas as pl
from jax.experimental.pallas import tpu as pltpu
from jax.experimental.pallas import tpu_sc as plsc

_NC, _NS, _L = 2, 16, 16
_NW = _NC * _NS
_MAX_SLIP = 0.4
_MAX_GUESS = 0.4
_T_INV = 1.0 / 50.0
_SW = 8


def _sigmoid(x):
    return 1.0 / (1.0 + jnp.exp(-x))


def _make_sc_kernel(B, H, S, C):
    RPW = B // _NW
    NCHUNK = RPW // C
    GRP = C // _L
    HB = H // _L
    OFF_W1 = 3 * H
    OFF_B1 = OFF_W1 + 48
    OFF_W2 = OFF_B1 + 16
    OFF_B2 = OFF_W2 + 16
    WPACK = OFF_B2 + 16

    mesh = plsc.VectorSubcoreMesh(core_axis_name="c", subcore_axis_name="s",
                                  num_cores=_NC, num_subcores=_NS)

    small = pltpu.VMEM((C, _SW), jnp.float32)
    @functools.partial(
        pl.kernel,
        out_type=jax.ShapeDtypeStruct((B,), jnp.float32),
        mesh=mesh,
        scratch_types=[
            pltpu.VMEM((C,), jnp.int32),
            pltpu.VMEM((C,), jnp.int32),
            pltpu.VMEM((C,), jnp.int32),
            pltpu.VMEM((C,), jnp.int32),
            pltpu.VMEM((C, H), jnp.float32),
            pltpu.VMEM((C, S, H), jnp.float32),
            pltpu.VMEM((C, H), jnp.float32),
            small, small, small,
            small, small,
            small, small,
            pltpu.VMEM((WPACK,), jnp.float32),
            pltpu.VMEM((_L, _L + 1), jnp.float32),
            pltpu.VMEM((_L, _L + 1), jnp.float32),
            pltpu.VMEM((3, _L), jnp.float32),
            pltpu.VMEM((C,), jnp.float32),
            pltpu.SemaphoreType.DMA,
        ],
        compiler_params=pltpu.CompilerParams(
            needs_layout_passes=False, use_tc_tiling_on_sc=False),
    )
    def k(user, item, knowledge, theta_t, a0_t, a1_t, a2_t, sl_t, gu_t,
          w0_t, w1_t, q_t, wpack, out,
          idx_u, idx_i, uridx_v, iridx_v, th_v, q_v, kn_v,
          a0_v, a1_v, a2_v, sl_v, gu_v, sw0_v, sw1_v,
          wp_v, st0_v, st1_v, fb_v, out_v, sem):
        wid = lax.axis_index("s") * _NC + lax.axis_index("c")
        pltpu.sync_copy(wpack, wp_v)
        lanes = lax.iota(jnp.int32, _L)
        z16 = jnp.zeros((_L,), jnp.int32)
        o16 = jnp.ones((_L,), jnp.int32)
        aw = [[wp_v[pl.ds(j * H + kk * _L, _L)] for kk in range(HB)]
              for j in range(3)]
        w1c = [wp_v[pl.ds(OFF_W1 + 16 * kk, 16)] for kk in range(3)]
        b1v = wp_v[pl.ds(OFF_B1, 16)]
        w2v = wp_v[pl.ds(OFF_W2, 16)]
        b2s = wp_v[pl.ds(OFF_B2, 16)][0]
        fzero = jnp.zeros((_L,), jnp.float32)

        def chunk_body(chunk, _):
            base = wid * RPW + chunk * C
            pltpu.sync_copy(user.at[pl.ds(base, C)], idx_u)
            pltpu.sync_copy(item.at[pl.ds(base, C)], idx_i)
            cps_a = (
                pltpu.async_copy(theta_t.at[idx_u], th_v, sem),
                pltpu.async_copy(q_t.at[idx_i], q_v, sem),
                pltpu.async_copy(knowledge.at[pl.ds(base, C)], kn_v, sem),
            )
            for g in range(GRP):
                sl = pl.ds(g * _L, _L)
                uridx_v[sl] = lax.shift_right_logical(idx_u[sl], 3)
                iridx_v[sl] = lax.shift_right_logical(idx_i[sl], 3)
            cps_b = (
                pltpu.async_copy(a0_t.at[uridx_v], a0_v, sem),
                pltpu.async_copy(a1_t.at[uridx_v], a1_v, sem),
                pltpu.async_copy(a2_t.at[uridx_v], a2_v, sem),
                pltpu.async_copy(sl_t.at[iridx_v], sl_v, sem),
                pltpu.async_copy(gu_t.at[iridx_v], gu_v, sem),
                pltpu.async_copy(w0_t.at[iridx_v], sw0_v, sem),
                pltpu.async_copy(w1_t.at[iridx_v], sw1_v, sem),
            )
            for cp in cps_a + cps_b:
                cp.wait()

            def group(g, _):
                r0 = g * _L
                ridx = r0 + lanes
                ou = jnp.bitwise_and(idx_u[pl.ds(r0, _L)], 7)
                oi = jnp.bitwise_and(idx_i[pl.ds(r0, _L)], 7)
                a0 = plsc.load_gather(a0_v, [ridx, ou])
                a1 = plsc.load_gather(a1_v, [ridx, ou])
                a2 = plsc.load_gather(a2_v, [ridx, ou])
                acc = fzero
                for j in range(16):
                    hj = w1c[0][j] * a0 + w1c[1][j] * a1 + w1c[2][j] * a2 + b1v[j]
                    acc = acc + w2v[j] * jnp.maximum(hj, 0.0)
                af = _sigmoid(acc + b2s)
                fb_v[0, pl.ds(0, _L)] = af * a0
                fb_v[1, pl.ds(0, _L)] = af * a1
                fb_v[2, pl.ds(0, _L)] = af * a2

                @plsc.parallel_loop(0, _L, 1, unroll=2)
                def _row(r):
                    row = r0 + r
                    rv = jnp.full((_L,), r, jnp.int32)
                    s0 = plsc.load_gather(fb_v, [z16, rv])
                    s1 = plsc.load_gather(fb_v, [o16, rv])
                    s2 = plsc.load_gather(fb_v, [z16 + 2, rv])
                    m0p = fzero
                    m1p = fzero
                    for kk in range(HB):
                        c0 = kk * _L
                        th = (th_v[row, pl.ds(c0, _L)]
                              + s0 * aw[0][kk] + s1 * aw[1][kk] + s2 * aw[2][kk])
                        sg = _sigmoid(th) - 0.5
                        kq = kn_v[row, pl.ds(c0, _L)] * sg
                        m0p = m0p + kq * _sigmoid(q_v[row, 0, pl.ds(c0, _L)])
                        m1p = m1p + kq * _sigmoid(q_v[row, 1, pl.ds(c0, _L)])
                    st0_v[r, pl.ds(0, _L)] = m0p
                    st1_v[r, pl.ds(0, _L)] = m1p

                m0 = fzero
                m1 = fzero
                for j in range(_L):
                    jj = jnp.full((_L,), j, jnp.int32)
                    m0 = m0 + plsc.load_gather(st0_v, [lanes, jj])
                    m1 = m1 + plsc.load_gather(st1_v, [lanes, jj])
                p0 = _sigmoid(m0 * _T_INV)
                p1 = _sigmoid(m1 * _T_INV)
                slip = _sigmoid(plsc.load_gather(sl_v, [ridx, oi])) * _MAX_SLIP
                guess = _sigmoid(plsc.load_gather(gu_v, [ridx, oi])) * _MAX_GUESS
                w0 = plsc.load_gather(sw0_v, [ridx, oi])
                w1 = plsc.load_gather(sw1_v, [ridx, oi])
                sp0 = _sigmoid(w0 - w1)
                span = 1.0 - slip - guess
                cc0 = guess + span * p0
                cc1 = guess + span * p1
                out_v[pl.ds(r0, _L)] = cc1 + sp0 * (cc0 - cc1)
                return 0

            lax.fori_loop(0, GRP, group, 0)
            pltpu.sync_copy(out_v, out.at[pl.ds(base, C)])
            return 0

        lax.fori_loop(0, NCHUNK, chunk_body, 0)

    return k


def kernel(user, item, knowledge, theta_table, affect_table, slip_table,
           guess_table, strategy_weights, strategy_q, affect_weight,
           W1, b1, W2, b2):
    B = user.shape[0]
    H = theta_table.shape[1]
    S = strategy_weights.shape[1]
    U = affect_table.shape[0]
    I = slip_table.shape[0]
    cols8 = lambda t, j: t[:, j].reshape(t.shape[0] // _SW, _SW)
    a0_t = cols8(affect_table, 0)
    a1_t = cols8(affect_table, 1)
    a2_t = cols8(affect_table, 2)
    sl_t = slip_table.reshape(I // _SW, _SW)
    gu_t = guess_table.reshape(I // _SW, _SW)
    w0_t = cols8(strategy_weights, 0)
    w1_t = cols8(strategy_weights, 1)
    wpack = jnp.concatenate([
        affect_weight.reshape(-1),
        W1.T.reshape(-1),
        b1.reshape(-1),
        W2.reshape(-1),
        b2.reshape(-1),
        jnp.zeros((15,), jnp.float32),
    ])
    k = _make_sc_kernel(B, H, S, 128)
    return k(user.astype(jnp.int32), item.astype(jnp.int32), knowledge,
             theta_table, a0_t, a1_t, a2_t, sl_t, gu_t, w0_t, w1_t,
             strategy_q, wpack)


# speedup vs baseline: 4.6508x; 1.1031x over previous
import functools

import jax
import jax.numpy as jnp
from jax import lax
from jax.experimental import pallas as pl
from jax.experimental.pallas import tpu as pltpu
from jax.experimental.pallas import tpu_sc as plsc

_NC, _NS, _L = 2, 16, 16
_NW = _NC * _NS
_MAX_SLIP = 0.4
_MAX_GUESS = 0.4
_T_INV = 1.0 / 50.0
_SW = 8


def _sigmoid(x):
    return 1.0 / (1.0 + jnp.exp(-x))


def _make_sc_kernel(B, H, S, C):
    RPW = B // _NW
    NCHUNK = RPW // C
    NPAIR = NCHUNK // 2
    GRP = C // _L
    HB = H // _L
    OFF_W1 = 3 * H
    OFF_B1 = OFF_W1 + 48
    OFF_W2 = OFF_B1 + 16
    OFF_B2 = OFF_W2 + 16
    WPACK = OFF_B2 + 16

    mesh = plsc.VectorSubcoreMesh(core_axis_name="c", subcore_axis_name="s",
                                  num_cores=_NC, num_subcores=_NS)

    bufset = (
        [pltpu.VMEM((C,), jnp.int32)] * 4
        + [pltpu.VMEM((C, H), jnp.float32),
           pltpu.VMEM((C, S, H), jnp.float32),
           pltpu.VMEM((C, H), jnp.float32)]
        + [pltpu.VMEM((C, _SW), jnp.float32)] * 7
        + [pltpu.VMEM((C,), jnp.float32)]
    )
    NBUF = len(bufset)

    @functools.partial(
        pl.kernel,
        out_type=jax.ShapeDtypeStruct((B,), jnp.float32),
        mesh=mesh,
        scratch_types=(
            bufset + bufset + [
                pltpu.VMEM((WPACK,), jnp.float32),
                pltpu.VMEM((_L, _L + 1), jnp.float32),
                pltpu.VMEM((_L, _L + 1), jnp.float32),
                pltpu.VMEM((3, _L), jnp.float32),
                pltpu.SemaphoreType.DMA,
                pltpu.SemaphoreType.DMA,
            ]
        ),
        compiler_params=pltpu.CompilerParams(
            needs_layout_passes=False, use_tc_tiling_on_sc=False),
    )
    def k(user, item, knowledge, theta_t, a0_t, a1_t, a2_t, sl_t, gu_t,
          w0_t, w1_t, q_t, wpack, out, *scratch):
        buf_a = scratch[:NBUF]
        buf_b = scratch[NBUF:2 * NBUF]
        wp_v, st0_v, st1_v, fb_v, sem_a, sem_b = scratch[2 * NBUF:]
        wid = lax.axis_index("s") * _NC + lax.axis_index("c")
        pltpu.sync_copy(wpack, wp_v)
        lanes = lax.iota(jnp.int32, _L)
        z16 = jnp.zeros((_L,), jnp.int32)
        o16 = jnp.ones((_L,), jnp.int32)
        aw = [[wp_v[pl.ds(j * H + kk * _L, _L)] for kk in range(HB)]
              for j in range(3)]
        w1c = [wp_v[pl.ds(OFF_W1 + 16 * kk, 16)] for kk in range(3)]
        b1v = wp_v[pl.ds(OFF_B1, 16)]
        w2v = wp_v[pl.ds(OFF_W2, 16)]
        b2s = wp_v[pl.ds(OFF_B2, 16)][0]
        fzero = jnp.zeros((_L,), jnp.float32)

        def descs(chunk, bufs, sem):
            (idx_u, idx_i, uridx, iridx, th_v, q_v, kn_v,
             a0v, a1v, a2v, slv, guv, w0v, w1v, out_v) = bufs
            base = wid * RPW + chunk * C
            return (
                (theta_t.at[idx_u], th_v),
                (q_t.at[idx_i], q_v),
                (knowledge.at[pl.ds(base, C)], kn_v),
                (a0_t.at[uridx], a0v),
                (a1_t.at[uridx], a1v),
                (a2_t.at[uridx], a2v),
                (sl_t.at[iridx], slv),
                (gu_t.at[iridx], guv),
                (w0_t.at[iridx], w0v),
                (w1_t.at[iridx], w1v),
            )

        def fire(chunk, bufs, sem):
            (idx_u, idx_i, uridx, iridx) = bufs[:4]
            base = wid * RPW + chunk * C
            pltpu.sync_copy(user.at[pl.ds(base, C)], idx_u)
            pltpu.sync_copy(item.at[pl.ds(base, C)], idx_i)
            for g in range(GRP):
                sl_ = pl.ds(g * _L, _L)
                uridx[sl_] = lax.shift_right_logical(idx_u[sl_], 3)
                iridx[sl_] = lax.shift_right_logical(idx_i[sl_], 3)
            for src, dst in descs(chunk, bufs, sem):
                pltpu.async_copy(src, dst, sem)

        def drain(chunk, bufs, sem):
            for src, dst in descs(chunk, bufs, sem):
                pltpu.make_async_copy(src, dst, sem).wait()

        def compute(chunk, bufs):
            (idx_u, idx_i, uridx, iridx, th_v, q_v, kn_v,
             a0v, a1v, a2v, slv, guv, w0v, w1v, out_v) = bufs
            base = wid * RPW + chunk * C

            def group(g, _):
                r0 = g * _L
                ridx = r0 + lanes
                ou = jnp.bitwise_and(idx_u[pl.ds(r0, _L)], 7)
                oi = jnp.bitwise_and(idx_i[pl.ds(r0, _L)], 7)
                a0 = plsc.load_gather(a0v, [ridx, ou])
                a1 = plsc.load_gather(a1v, [ridx, ou])
                a2 = plsc.load_gather(a2v, [ridx, ou])
                acc = fzero
                for j in range(16):
                    hj = w1c[0][j] * a0 + w1c[1][j] * a1 + w1c[2][j] * a2 + b1v[j]
                    acc = acc + w2v[j] * jnp.maximum(hj, 0.0)
                af = _sigmoid(acc + b2s)
                fb_v[0, pl.ds(0, _L)] = af * a0
                fb_v[1, pl.ds(0, _L)] = af * a1
                fb_v[2, pl.ds(0, _L)] = af * a2

                @plsc.parallel_loop(0, _L, 1, unroll=2)
                def _row(r):
                    row = r0 + r
                    rv = jnp.full((_L,), r, jnp.int32)
                    s0 = plsc.load_gather(fb_v, [z16, rv])
                    s1 = plsc.load_gather(fb_v, [o16, rv])
                    s2 = plsc.load_gather(fb_v, [z16 + 2, rv])
                    m0p = fzero
                    m1p = fzero
                    for kk in range(HB):
                        c0 = kk * _L
                        th = (th_v[row, pl.ds(c0, _L)]
                              + s0 * aw[0][kk] + s1 * aw[1][kk] + s2 * aw[2][kk])
                        sg = _sigmoid(th) - 0.5
                        kq = kn_v[row, pl.ds(c0, _L)] * sg
                        m0p = m0p + kq * _sigmoid(q_v[row, 0, pl.ds(c0, _L)])
                        m1p = m1p + kq * _sigmoid(q_v[row, 1, pl.ds(c0, _L)])
                    st0_v[r, pl.ds(0, _L)] = m0p
                    st1_v[r, pl.ds(0, _L)] = m1p

                m0 = fzero
                m1 = fzero
                for j in range(_L):
                    jj = jnp.full((_L,), j, jnp.int32)
                    m0 = m0 + plsc.load_gather(st0_v, [lanes, jj])
                    m1 = m1 + plsc.load_gather(st1_v, [lanes, jj])
                p0 = _sigmoid(m0 * _T_INV)
                p1 = _sigmoid(m1 * _T_INV)
                slip = _sigmoid(plsc.load_gather(slv, [ridx, oi])) * _MAX_SLIP
                guess = _sigmoid(plsc.load_gather(guv, [ridx, oi])) * _MAX_GUESS
                w0 = plsc.load_gather(w0v, [ridx, oi])
                w1 = plsc.load_gather(w1v, [ridx, oi])
                sp0 = _sigmoid(w0 - w1)
                span = 1.0 - slip - guess
                cc0 = guess + span * p0
                cc1 = guess + span * p1
                out_v[pl.ds(r0, _L)] = cc1 + sp0 * (cc0 - cc1)
                return 0

            lax.fori_loop(0, GRP, group, 0)
            pltpu.sync_copy(out_v, out.at[pl.ds(base, C)])

        fire(0, buf_a, sem_a)

        def pair_body(i, _):
            c0 = 2 * i
            c1 = c0 + 1
            fire(c1, buf_b, sem_b)
            drain(c0, buf_a, sem_a)
            compute(c0, buf_a)

            @pl.when(c0 + 2 < NCHUNK)
            def _fire_next():
                fire(c0 + 2, buf_a, sem_a)

            drain(c1, buf_b, sem_b)
            compute(c1, buf_b)
            return 0

        lax.fori_loop(0, NPAIR, pair_body, 0)

    return k


def kernel(user, item, knowledge, theta_table, affect_table, slip_table,
           guess_table, strategy_weights, strategy_q, affect_weight,
           W1, b1, W2, b2):
    B = user.shape[0]
    H = theta_table.shape[1]
    S = strategy_weights.shape[1]
    I = slip_table.shape[0]
    cols8 = lambda t, j: t[:, j].reshape(t.shape[0] // _SW, _SW)
    a0_t = cols8(affect_table, 0)
    a1_t = cols8(affect_table, 1)
    a2_t = cols8(affect_table, 2)
    sl_t = slip_table.reshape(I // _SW, _SW)
    gu_t = guess_table.reshape(I // _SW, _SW)
    w0_t = cols8(strategy_weights, 0)
    w1_t = cols8(strategy_weights, 1)
    wpack = jnp.concatenate([
        affect_weight.reshape(-1),
        W1.T.reshape(-1),
        b1.reshape(-1),
        W2.reshape(-1),
        b2.reshape(-1),
        jnp.zeros((15,), jnp.float32),
    ])
    k = _make_sc_kernel(B, H, S, 64)
    return k(user.astype(jnp.int32), item.astype(jnp.int32), knowledge,
             theta_table, a0_t, a1_t, a2_t, sl_t, gu_t, w0_t, w1_t,
             strategy_q, wpack)
